# baseline (speedup 1.0000x reference)
import functools

import jax
import jax.numpy as jnp
import numpy as np
from jax.experimental import pallas as pl

RATIO = 0.25
R = 0.1
NUM_FREQS = 4
MAX_NEIGHBORS = 512
N_NODES = 10000
D_FEAT = 128
HID = 256

N_PAD = 10240
M = 2500
M_PAD = 2560
QT = 8
ET = QT * MAX_NEIGHBORS
D_GE = 136
D_PE = 32


def _fps_body(pos_ref, idx_ref):
    px = pos_ref[0:8, :]
    py = pos_ref[8:16, :]
    pz = pos_ref[16:24, :]
    sub = jax.lax.broadcasted_iota(jnp.int32, (8, 1280), 0)
    lane = jax.lax.broadcasted_iota(jnp.int32, (8, 1280), 1)
    jidx = sub * 1280 + lane
    real = jidx < N_NODES
    dist0 = jnp.where(real, jnp.inf, -jnp.inf).astype(jnp.float32)

    osub = jax.lax.broadcasted_iota(jnp.int32, (8, 320), 0)
    olane = jax.lax.broadcasted_iota(jnp.int32, (8, 320), 1)
    opos = osub * 320 + olane
    idx0 = jnp.zeros((8, 320), jnp.int32)

    def body(i, carry):
        idxv, dist, cur = carry
        idxv = jnp.where(opos == i, cur, idxv)
        sel = jidx == cur
        cx = jnp.sum(jnp.where(sel, px, 0.0))
        cy = jnp.sum(jnp.where(sel, py, 0.0))
        cz = jnp.sum(jnp.where(sel, pz, 0.0))
        dx = px - cx
        dy = py - cy
        dz = pz - cz
        d = (dx * dx + dy * dy) + dz * dz
        dist = jnp.minimum(dist, d)
        mx = jnp.max(dist)
        cur = jnp.min(jnp.where(dist == mx, jidx, N_PAD)).astype(jnp.int32)
        return idxv, dist, cur

    idxv, _, _ = jax.lax.fori_loop(
        0, M, body, (idx0, dist0, jnp.int32(0)))
    idx_ref[...] = idxv


def _fps(pos):
    posp = jnp.pad(pos, ((0, N_PAD - N_NODES), (0, 0)))
    pos24 = jnp.concatenate(
        [posp[:, 0].reshape(8, 1280),
         posp[:, 1].reshape(8, 1280),
         posp[:, 2].reshape(8, 1280)], axis=0)
    idx = pl.pallas_call(
        _fps_body,
        out_shape=jax.ShapeDtypeStruct((8, 320), jnp.int32),
    )(pos24)
    return idx.reshape(-1)[:M]


def _scores_body(q_ref, pt_ref, s_ref):
    qx = q_ref[:, 0:1]
    qy = q_ref[:, 1:2]
    qz = q_ref[:, 2:3]
    q2 = (qx * qx + qy * qy) + qz * qz
    px = pt_ref[0:1, :]
    py = pt_ref[1:2, :]
    pz = pt_ref[2:3, :]
    p2 = (px * px + py * py) + pz * pz
    m = jnp.dot(q_ref[...], pt_ref[...],
                precision=jax.lax.Precision.HIGHEST,
                preferred_element_type=jnp.float32)
    d2 = (q2 + p2) - 2.0 * m
    jcand = jax.lax.broadcasted_iota(jnp.int32, d2.shape, 1)
    mask = (d2 < R * R) & (jcand < N_NODES)
    s_ref[...] = jnp.where(mask, jcand, N_NODES)


def _scores(pos_dst, pos):
    qp = jnp.pad(pos_dst, ((0, M_PAD - M), (0, 5)))
    posp = jnp.pad(pos.T, ((0, 5), (0, N_PAD - N_NODES)))
    return pl.pallas_call(
        _scores_body,
        grid=(M_PAD // 128,),
        in_specs=[
            pl.BlockSpec((128, 8), lambda i: (i, 0)),
            pl.BlockSpec((8, N_PAD), lambda i: (0, 0)),
        ],
        out_specs=pl.BlockSpec((128, N_PAD), lambda i: (i, 0)),
        out_shape=jax.ShapeDtypeStruct((M_PAD, N_PAD), jnp.int32),
    )(qp, posp)[:M]


def _mlp_body(ge_ref, pe_ref, w1a_ref, w1b_ref, b1_ref, w2_ref, b2_ref,
              g1_ref, gb1_ref, g2_ref, gb2_ref, out_ref):
    ge = ge_ref[...]
    pe = pe_ref[...]
    w = pe[:, D_PE - 1:D_PE].astype(jnp.float32)
    h = (jnp.dot(ge[:, :D_FEAT], w1a_ref[...].astype(jnp.bfloat16),
                 preferred_element_type=jnp.float32) +
         jnp.dot(pe, w1b_ref[...].astype(jnp.bfloat16),
                 preferred_element_type=jnp.float32))
    h = jax.nn.gelu(h + b1_ref[0:1, :], approximate=True)
    h = jnp.dot(h.astype(jnp.bfloat16), w2_ref[...].astype(jnp.bfloat16),
                preferred_element_type=jnp.float32)
    h = (h + b2_ref[0:1, :]) * w
    sums = []
    cnts = []
    for q in range(QT):
        lo = q * MAX_NEIGHBORS
        hi = lo + MAX_NEIGHBORS
        sums.append(jnp.sum(h[lo:hi, :], axis=0, keepdims=True))
        cnts.append(jnp.sum(w[lo:hi, :], axis=0, keepdims=True))
    summed = jnp.concatenate(sums, axis=0)
    counts = jnp.concatenate(cnts, axis=0)
    out = summed / jnp.maximum(counts, 1.0)
    out = jnp.dot(out, g1_ref[...], preferred_element_type=jnp.float32)
    out = jax.nn.gelu(out + gb1_ref[0:1, :], approximate=True)
    out = jnp.dot(out, g2_ref[...], preferred_element_type=jnp.float32)
    out_ref[...] = out + gb2_ref[0:1, :]


def _mlp(ge, pe, W1a, W1b, b1, W2, b2, G1, gb1, G2, gb2):
    def rep(b):
        return jnp.broadcast_to(b.reshape(1, HID), (8, HID))

    const = lambda i: (0, 0)
    return pl.pallas_call(
        _mlp_body,
        grid=(M_PAD // QT,),
        in_specs=[
            pl.BlockSpec((ET, D_GE), lambda i: (i, 0)),
            pl.BlockSpec((ET, D_PE), lambda i: (i, 0)),
            pl.BlockSpec((D_FEAT, HID), const),
            pl.BlockSpec((D_PE, HID), const),
            pl.BlockSpec((8, HID), const),
            pl.BlockSpec((HID, HID), const),
            pl.BlockSpec((8, HID), const),
            pl.BlockSpec((HID, HID), const),
            pl.BlockSpec((8, HID), const),
            pl.BlockSpec((HID, HID), const),
            pl.BlockSpec((8, HID), const),
        ],
        out_specs=pl.BlockSpec((QT, HID), lambda i: (i, 0)),
        out_shape=jax.ShapeDtypeStruct((M_PAD, HID), jnp.float32),
    )(ge, pe, W1a, W1b, rep(b1), W2, rep(b2), G1, rep(gb1), G2, rep(gb2))


def _pe27(rel):
    freqs = np.linspace(1.0, 2.0 ** (NUM_FREQS - 1), NUM_FREQS)
    outs = [rel]
    for f in freqs:
        outs.append(jnp.sin(rel * float(f)))
        outs.append(jnp.cos(rel * float(f)))
    return jnp.concatenate(outs, axis=-1)


@jax.jit
def kernel(x, pos, batch, W1, b1, W2, b2, G1, gb1, G2, gb2):
    idx = _fps(pos)
    pos_dst = pos[idx]
    scores = _scores(pos_dst, pos)
    maskv = scores < N_NODES
    C = jnp.cumsum(maskv.astype(jnp.int32), axis=1)
    total = C[:, -1]
    C16 = C.astype(jnp.int16)
    T16 = C16[:, 15::16]
    s1 = jnp.arange(1, MAX_NEIGHBORS + 1, dtype=jnp.int32)
    s116 = s1.astype(jnp.int16)
    g = jnp.sum((T16[:, None, :] < s116[None, :, None]).astype(jnp.int32),
                axis=2)
    lo = jnp.minimum(g, N_PAD // 16 - 1) * 16
    for span in (8, 4, 2, 1):
        v = jnp.take_along_axis(C16, lo + (span - 1), axis=1)
        lo = jnp.where(v < s116[None, :], lo + span, lo)
    valid = s1[None, :] <= total[:, None]
    col = jnp.where(valid, lo, 0)

    colf = col.reshape(-1)
    pos_b = jax.lax.bitcast_convert_type(
        pos, jnp.bfloat16).reshape(N_NODES, 6)
    xcat = jnp.concatenate(
        [x.astype(jnp.bfloat16), pos_b,
         jnp.zeros((N_NODES, D_GE - 134), jnp.bfloat16)], axis=1)
    ge = xcat[colf]
    posg = jax.lax.bitcast_convert_type(
        ge[:, 128:134].reshape(-1, 3, 2), jnp.float32)
    rel = (posg - jnp.repeat(pos_dst, MAX_NEIGHBORS, axis=0)) / R
    pe = jnp.concatenate(
        [_pe27(rel).astype(jnp.bfloat16),
         jnp.zeros((M * MAX_NEIGHBORS, D_PE - 28), jnp.bfloat16),
         valid.reshape(-1, 1).astype(jnp.bfloat16)], axis=1)
    epad = ((0, (M_PAD - M) * MAX_NEIGHBORS), (0, 0))
    ge = jnp.pad(ge, epad)
    pe = jnp.pad(pe, epad)

    W1a = W1[:D_FEAT]
    W1b = jnp.pad(W1[D_FEAT:], ((0, D_PE - (155 - D_FEAT)), (0, 0)))
    out = _mlp(ge, pe, W1a, W1b, b1, W2, b2, G1, gb1, G2, gb2)[:M]
    return out, pos_dst, batch[idx]


# speedup vs baseline: 1.0397x; 1.0397x over previous
import functools

import jax
import jax.numpy as jnp
import numpy as np
from jax.experimental import pallas as pl

RATIO = 0.25
R = 0.1
NUM_FREQS = 4
MAX_NEIGHBORS = 512
N_NODES = 10000
D_FEAT = 128
HID = 256

N_PAD = 10240
M = 2500
M_PAD = 2560
QT = 8
ET = QT * MAX_NEIGHBORS
D_GE = 136
D_PE = 32


def _fps_body(pos_ref, idx_ref):
    px = pos_ref[0:8, :]
    py = pos_ref[8:16, :]
    pz = pos_ref[16:24, :]
    sub = jax.lax.broadcasted_iota(jnp.int32, (8, 1280), 0)
    lane = jax.lax.broadcasted_iota(jnp.int32, (8, 1280), 1)
    jidx = sub * 1280 + lane
    real = jidx < N_NODES
    dist0 = jnp.where(real, jnp.inf, -jnp.inf).astype(jnp.float32)

    osub = jax.lax.broadcasted_iota(jnp.int32, (8, 320), 0)
    olane = jax.lax.broadcasted_iota(jnp.int32, (8, 320), 1)
    opos = osub * 320 + olane
    idx0 = jnp.zeros((8, 320), jnp.int32)

    def body(i, carry):
        idxv, dist, cur = carry
        idxv = jnp.where(opos == i, cur, idxv)
        sel = jidx == cur
        cx = jnp.sum(jnp.where(sel, px, 0.0))
        cy = jnp.sum(jnp.where(sel, py, 0.0))
        cz = jnp.sum(jnp.where(sel, pz, 0.0))
        dx = px - cx
        dy = py - cy
        dz = pz - cz
        d = (dx * dx + dy * dy) + dz * dz
        dist = jnp.minimum(dist, d)
        mx = jnp.max(dist)
        cur = jnp.min(jnp.where(dist == mx, jidx, N_PAD)).astype(jnp.int32)
        return idxv, dist, cur

    idxv, _, _ = jax.lax.fori_loop(
        0, M, body, (idx0, dist0, jnp.int32(0)))
    idx_ref[...] = idxv


def _fps(pos):
    posp = jnp.pad(pos, ((0, N_PAD - N_NODES), (0, 0)))
    pos24 = jnp.concatenate(
        [posp[:, 0].reshape(8, 1280),
         posp[:, 1].reshape(8, 1280),
         posp[:, 2].reshape(8, 1280)], axis=0)
    idx = pl.pallas_call(
        _fps_body,
        out_shape=jax.ShapeDtypeStruct((8, 320), jnp.int32),
    )(pos24)
    return idx.reshape(-1)[:M]


def _scores_body(q_ref, pt_ref, s_ref):
    qx = q_ref[:, 0:1]
    qy = q_ref[:, 1:2]
    qz = q_ref[:, 2:3]
    q2 = (qx * qx + qy * qy) + qz * qz
    px = pt_ref[0:1, :]
    py = pt_ref[1:2, :]
    pz = pt_ref[2:3, :]
    p2 = (px * px + py * py) + pz * pz
    m = jnp.dot(q_ref[...], pt_ref[...],
                precision=jax.lax.Precision.HIGHEST,
                preferred_element_type=jnp.float32)
    d2 = (q2 + p2) - 2.0 * m
    jcand = jax.lax.broadcasted_iota(jnp.int32, d2.shape, 1)
    mask = (d2 < R * R) & (jcand < N_NODES)
    s_ref[...] = jnp.where(mask, jcand, N_NODES)


def _scores(pos_dst, pos):
    qp = jnp.pad(pos_dst, ((0, M_PAD - M), (0, 5)))
    posp = jnp.pad(pos.T, ((0, 5), (0, N_PAD - N_NODES)))
    return pl.pallas_call(
        _scores_body,
        grid=(M_PAD // 128,),
        in_specs=[
            pl.BlockSpec((128, 8), lambda i: (i, 0)),
            pl.BlockSpec((8, N_PAD), lambda i: (0, 0)),
        ],
        out_specs=pl.BlockSpec((128, N_PAD), lambda i: (i, 0)),
        out_shape=jax.ShapeDtypeStruct((M_PAD, N_PAD), jnp.int32),
    )(qp, posp)[:M]


def _mlp_body(ge_ref, pe_ref, w1a_ref, w1b_ref, b1_ref, w2_ref, b2_ref,
              g1_ref, gb1_ref, g2_ref, gb2_ref, out_ref):
    ge = ge_ref[...]
    pe = pe_ref[...]
    w = pe[:, D_PE - 1:D_PE].astype(jnp.float32)
    h = (jnp.dot(ge[:, :D_FEAT], w1a_ref[...].astype(jnp.bfloat16),
                 preferred_element_type=jnp.float32) +
         jnp.dot(pe, w1b_ref[...].astype(jnp.bfloat16),
                 preferred_element_type=jnp.float32))
    h = jax.nn.gelu(h + b1_ref[0:1, :], approximate=True)
    h = jnp.dot(h.astype(jnp.bfloat16), w2_ref[...].astype(jnp.bfloat16),
                preferred_element_type=jnp.float32)
    h = (h + b2_ref[0:1, :]) * w
    sums = []
    cnts = []
    for q in range(QT):
        lo = q * MAX_NEIGHBORS
        hi = lo + MAX_NEIGHBORS
        sums.append(jnp.sum(h[lo:hi, :], axis=0, keepdims=True))
        cnts.append(jnp.sum(w[lo:hi, :], axis=0, keepdims=True))
    summed = jnp.concatenate(sums, axis=0)
    counts = jnp.concatenate(cnts, axis=0)
    out = summed / jnp.maximum(counts, 1.0)
    out = jnp.dot(out, g1_ref[...], preferred_element_type=jnp.float32)
    out = jax.nn.gelu(out + gb1_ref[0:1, :], approximate=True)
    out = jnp.dot(out, g2_ref[...], preferred_element_type=jnp.float32)
    out_ref[...] = out + gb2_ref[0:1, :]


def _mlp(ge, pe, W1a, W1b, b1, W2, b2, G1, gb1, G2, gb2):
    def rep(b):
        return jnp.broadcast_to(b.reshape(1, HID), (8, HID))

    const = lambda i: (0, 0)
    return pl.pallas_call(
        _mlp_body,
        grid=(M_PAD // QT,),
        in_specs=[
            pl.BlockSpec((ET, D_GE), lambda i: (i, 0)),
            pl.BlockSpec((ET, D_PE), lambda i: (i, 0)),
            pl.BlockSpec((D_FEAT, HID), const),
            pl.BlockSpec((D_PE, HID), const),
            pl.BlockSpec((8, HID), const),
            pl.BlockSpec((HID, HID), const),
            pl.BlockSpec((8, HID), const),
            pl.BlockSpec((HID, HID), const),
            pl.BlockSpec((8, HID), const),
            pl.BlockSpec((HID, HID), const),
            pl.BlockSpec((8, HID), const),
        ],
        out_specs=pl.BlockSpec((QT, HID), lambda i: (i, 0)),
        out_shape=jax.ShapeDtypeStruct((M_PAD, HID), jnp.float32),
    )(ge, pe, W1a, W1b, rep(b1), W2, rep(b2), G1, rep(gb1), G2, rep(gb2))


def _pe27(rel):
    freqs = np.linspace(1.0, 2.0 ** (NUM_FREQS - 1), NUM_FREQS)
    outs = [rel]
    for f in freqs:
        outs.append(jnp.sin(rel * float(f)))
        outs.append(jnp.cos(rel * float(f)))
    return jnp.concatenate(outs, axis=-1)


@jax.jit
def kernel(x, pos, batch, W1, b1, W2, b2, G1, gb1, G2, gb2):
    idx = _fps(pos)
    pos_dst = pos[idx]
    scores = _scores(pos_dst, pos)
    maskv = scores < N_NODES
    C = jnp.cumsum(maskv.astype(jnp.int32), axis=1)
    total = C[:, -1]
    T = C[:, 127::128]
    C16 = C.astype(jnp.int16)
    s1 = jnp.arange(1, MAX_NEIGHBORS + 1, dtype=jnp.int32)
    s116 = s1.astype(jnp.int16)
    g = jnp.sum((T[:, None, :] < s1[None, :, None]).astype(jnp.int32), axis=2)
    lo = jnp.minimum(g, N_PAD // 128 - 1) * 128
    for span in (64, 32, 16, 8, 4, 2, 1):
        v = jnp.take_along_axis(C16, lo + (span - 1), axis=1)
        lo = jnp.where(v < s116[None, :], lo + span, lo)
    valid = s1[None, :] <= total[:, None]
    col = jnp.where(valid, lo, 0)

    colf = col.reshape(-1)
    pos_b = jax.lax.bitcast_convert_type(
        pos, jnp.bfloat16).reshape(N_NODES, 6)
    xcat = jnp.concatenate(
        [x.astype(jnp.bfloat16), pos_b,
         jnp.zeros((N_NODES, D_GE - 134), jnp.bfloat16)], axis=1)
    ge = xcat[colf]
    posg = jax.lax.bitcast_convert_type(
        ge[:, 128:134].reshape(-1, 3, 2), jnp.float32)
    rel = (posg - jnp.repeat(pos_dst, MAX_NEIGHBORS, axis=0)) / R
    pe = jnp.concatenate(
        [_pe27(rel).astype(jnp.bfloat16),
         jnp.zeros((M * MAX_NEIGHBORS, D_PE - 28), jnp.bfloat16),
         valid.reshape(-1, 1).astype(jnp.bfloat16)], axis=1)
    epad = ((0, (M_PAD - M) * MAX_NEIGHBORS), (0, 0))
    ge = jnp.pad(ge, epad)
    pe = jnp.pad(pe, epad)

    W1a = W1[:D_FEAT]
    W1b = jnp.pad(W1[D_FEAT:], ((0, D_PE - (155 - D_FEAT)), (0, 0)))
    out = _mlp(ge, pe, W1a, W1b, b1, W2, b2, G1, gb1, G2, gb2)[:M]
    return out, pos_dst, batch[idx]
